# Initial kernel scaffold; baseline (speedup 1.0000x reference)
import jax
import jax.numpy as jnp
from jax.experimental import pallas as pl


def kernel(x, edge_index, edge_vectors, g_weight, mu, sigma, root_weight, bias):
    raise NotImplementedError("write your pallas kernel here")


# speedup vs baseline: 1.7949x; 1.7949x over previous
import jax
import jax.numpy as jnp
from jax import lax
from jax.experimental import pallas as pl
from jax.experimental.pallas import tpu as pltpu
from jax.experimental.pallas import tpu_sc as plsc

N_NODES = 10000
N_EDGES = 320000
IN_CH = 128
OUT_CH = 128
DIM = 4
K = 8
EPS = 1e-15

L = 16
NC = 2
NS = 16
NW = NC * NS
EPW = N_EDGES // NW
BE = 16
NIT = EPW // BE
KW = K * OUT_CH


def _mm_body(x_ref, w_ref, o_ref):
    o_ref[...] = jnp.dot(x_ref[...], w_ref[...],
                         preferred_element_type=jnp.float32)


def _xg_matmul(x, g_weight):
    blk = 1000
    grid = (N_NODES // blk,)
    return pl.pallas_call(
        _mm_body,
        grid=grid,
        in_specs=[
            pl.BlockSpec((blk, IN_CH), lambda i: (i, 0)),
            pl.BlockSpec((IN_CH, KW), lambda i: (0, 0)),
        ],
        out_specs=pl.BlockSpec((blk, KW), lambda i: (i, 0)),
        out_shape=jax.ShapeDtypeStruct((N_NODES, KW), jnp.float32),
    )(x, g_weight)


def _gauss_body(ev_ref, muT_ref, sgT_ref, o_ref):
    ev = ev_ref[...]
    muT = muT_ref[...]
    sgT = sgT_ref[...]
    w = 0.5 / (EPS + sgT * sgT)
    lin = jnp.zeros((ev.shape[0], L), jnp.float32)
    for d in range(DIM):
        diff = ev[:, d:d + 1] - muT[d:d + 1, :]
        lin = lin - diff * diff * w[d:d + 1, :]
    o_ref[...] = jnp.exp(lin)


def _gauss_weights(edge_vectors, muT, sgT):
    blk = 8000
    grid = (N_EDGES // blk,)
    return pl.pallas_call(
        _gauss_body,
        grid=grid,
        in_specs=[
            pl.BlockSpec((blk, DIM), lambda i: (i, 0)),
            pl.BlockSpec((DIM, L), lambda i: (0, 0)),
            pl.BlockSpec((DIM, L), lambda i: (0, 0)),
        ],
        out_specs=pl.BlockSpec((blk, L), lambda i: (i, 0)),
        out_shape=jax.ShapeDtypeStruct((N_EDGES, L), jnp.float32),
    )(edge_vectors, muT, sgT)


def _lane_bcast(v, lane):
    idx = jnp.full((L, 1), lane, dtype=jnp.int32)
    dn = lax.GatherDimensionNumbers(offset_dims=(),
                                    collapsed_slice_dims=(0,),
                                    start_index_map=(0,))
    return lax.gather(v, idx, dn, (1,),
                      mode=lax.GatherScatterMode.PROMISE_IN_BOUNDS)


def _sc_edge_body(xg_hbm, g_hbm, src_hbm, dst_hbm, zero_hbm, out_hbm,
                  cnt_hbm, acc, src_v, dst_v, g_v, rows_v, msg_v,
                  cnt_v, sem):
    c = lax.axis_index("c")
    s = lax.axis_index("s")
    wid = s * NC + c
    rps = 624
    tail = N_NODES - NS * rps
    pltpu.sync_copy(zero_hbm.at[pl.ds(s * rps, rps)],
                    acc.at[pl.ds(s * rps, rps)])

    @pl.when(s == 0)
    def _zero_tail():
        pltpu.sync_copy(zero_hbm.at[pl.ds(NS * rps, tail)],
                        acc.at[pl.ds(NS * rps, tail)])

    zeros16 = jnp.zeros((L,), jnp.float32)
    ones16 = jnp.ones((L,), jnp.float32)
    lane0 = lax.iota(jnp.int32, L) == 0

    def zero_cnt(i, carry):
        cnt_v[pl.ds(i * L, L)] = zeros16
        return carry

    lax.fori_loop(0, N_NODES // L, zero_cnt, 0)
    plsc.subcore_barrier()

    def batch_body(i, carry):
        base = wid * EPW + i * BE
        pltpu.sync_copy(src_hbm.at[pl.ds(base, BE)], src_v)
        pltpu.sync_copy(dst_hbm.at[pl.ds(base, BE)], dst_v)
        pltpu.sync_copy(g_hbm.at[pl.ds(base, BE)], g_v)
        pltpu.async_copy(xg_hbm.at[src_v], rows_v, sem).wait()

        def edge_body(e, carry2):
            gv = g_v[e, :]
            msgs = [jnp.zeros((L,), jnp.float32) for _ in range(8)]
            for k in range(K):
                gk = _lane_bcast(gv, k)
                for j in range(8):
                    msgs[j] = msgs[j] + gk * rows_v[e, pl.ds(k * 128 + j * L, L)]
            for j in range(8):
                msg_v[e, pl.ds(j * L, L)] = msgs[j]
            dwin = dst_v[...]
            dlane = _lane_bcast(dwin, e)
            plsc.addupdate_scatter(cnt_v, [dlane], ones16, mask=lane0)
            return carry2

        lax.fori_loop(0, BE, edge_body, 0)
        pltpu.sync_copy(msg_v, acc.at[dst_v], add=True)
        return carry

    lax.fori_loop(0, NIT, batch_body, 0)
    plsc.subcore_barrier()
    pltpu.sync_copy(acc.at[pl.ds(s * rps, rps)],
                    out_hbm.at[c, pl.ds(s * rps, rps)])

    @pl.when(s == 0)
    def _out_tail():
        pltpu.sync_copy(acc.at[pl.ds(NS * rps, tail)],
                        out_hbm.at[c, pl.ds(NS * rps, tail)])

    pltpu.sync_copy(cnt_v, cnt_hbm.at[wid])


def _sc_edge(xg, gauss, src, dst, zero):
    mesh = plsc.VectorSubcoreMesh(core_axis_name="c", subcore_axis_name="s")
    fn = pl.kernel(
        _sc_edge_body,
        out_type=(
            jax.ShapeDtypeStruct((NC, N_NODES, OUT_CH), jnp.float32),
            jax.ShapeDtypeStruct((NW, N_NODES), jnp.float32),
        ),
        mesh=mesh,
        compiler_params=pltpu.CompilerParams(needs_layout_passes=False),
        scratch_types=[
            pltpu.VMEM_SHARED((N_NODES, OUT_CH), jnp.float32),
            pltpu.VMEM((BE,), jnp.int32),
            pltpu.VMEM((BE,), jnp.int32),
            pltpu.VMEM((BE, L), jnp.float32),
            pltpu.VMEM((BE, KW), jnp.float32),
            pltpu.VMEM((BE, OUT_CH), jnp.float32),
            pltpu.VMEM((N_NODES,), jnp.float32),
            pltpu.SemaphoreType.DMA,
        ],
    )
    return fn(xg, gauss, src, dst, zero)


def _combine_body(acc_ref, cnt_ref, x_ref, rw_ref, b_ref, o_ref):
    sacc = acc_ref[0] + acc_ref[1]
    csum = jnp.sum(cnt_ref[...], axis=1, keepdims=True)
    mean = sacc / jnp.maximum(csum, 1.0)
    root = jnp.dot(x_ref[...], rw_ref[...], preferred_element_type=jnp.float32)
    o_ref[...] = mean + root + b_ref[...]


def _combine(acc, cnt, x, root_weight, bias2d):
    blk = 1000
    grid = (N_NODES // blk,)
    return pl.pallas_call(
        _combine_body,
        grid=grid,
        in_specs=[
            pl.BlockSpec((NC, blk, OUT_CH), lambda i: (0, i, 0)),
            pl.BlockSpec((blk, NW), lambda i: (i, 0)),
            pl.BlockSpec((blk, IN_CH), lambda i: (i, 0)),
            pl.BlockSpec((IN_CH, OUT_CH), lambda i: (0, 0)),
            pl.BlockSpec((1, OUT_CH), lambda i: (0, 0)),
        ],
        out_specs=pl.BlockSpec((blk, OUT_CH), lambda i: (i, 0)),
        out_shape=jax.ShapeDtypeStruct((N_NODES, OUT_CH), jnp.float32),
    )(acc, cnt, x, root_weight, bias2d)


def kernel(x, edge_index, edge_vectors, g_weight, mu, sigma, root_weight, bias):
    src = edge_index[0].astype(jnp.int32)
    dst = edge_index[1].astype(jnp.int32)
    mu16 = jnp.concatenate([mu, jnp.full((L - K, DIM), 1e16, mu.dtype)], axis=0)
    sg16 = jnp.concatenate([sigma, jnp.ones((L - K, DIM), sigma.dtype)], axis=0)
    muT = mu16.T
    sgT = sg16.T
    zero = jnp.zeros((N_NODES, OUT_CH), jnp.float32)
    bias2d = bias.reshape(1, OUT_CH)

    xg = _xg_matmul(x, g_weight)
    gauss = _gauss_weights(edge_vectors, muT, sgT)
    acc, cnt = _sc_edge(xg, gauss, src, dst, zero)
    return _combine(acc, cnt.T, x, root_weight, bias2d)


# speedup vs baseline: 2.4117x; 1.3436x over previous
import jax
import jax.numpy as jnp
from jax import lax
from jax.experimental import pallas as pl
from jax.experimental.pallas import tpu as pltpu
from jax.experimental.pallas import tpu_sc as plsc

N_NODES = 10000
N_EDGES = 320000
IN_CH = 128
OUT_CH = 128
DIM = 4
K = 8
EPS = 1e-15

L = 16
NC = 2
NS = 16
NW = NC * NS
EPW = N_EDGES // NW
BE = 8
CH = 10
CE = CH * BE
NCH = EPW // CE
KW = K * OUT_CH


def _mm_body(x_ref, w_ref, o_ref):
    o_ref[...] = jnp.dot(x_ref[...], w_ref[...],
                         preferred_element_type=jnp.float32)


def _xg_matmul(x, g_weight):
    blk = 1000
    grid = (N_NODES // blk,)
    return pl.pallas_call(
        _mm_body,
        grid=grid,
        in_specs=[
            pl.BlockSpec((blk, IN_CH), lambda i: (i, 0)),
            pl.BlockSpec((IN_CH, KW), lambda i: (0, 0)),
        ],
        out_specs=pl.BlockSpec((blk, KW), lambda i: (i, 0)),
        out_shape=jax.ShapeDtypeStruct((N_NODES, KW), jnp.float32),
    )(x, g_weight)


def _gauss_body(ev_ref, muT_ref, sgT_ref, o_ref):
    ev = ev_ref[...]
    muT = muT_ref[...]
    sgT = sgT_ref[...]
    w = 0.5 / (EPS + sgT * sgT)
    lin = jnp.zeros((ev.shape[0], L), jnp.float32)
    for d in range(DIM):
        diff = ev[:, d:d + 1] - muT[d:d + 1, :]
        lin = lin - diff * diff * w[d:d + 1, :]
    o_ref[...] = jnp.exp(lin)


def _gauss_weights(edge_vectors, muT, sgT):
    blk = 8000
    grid = (N_EDGES // blk,)
    return pl.pallas_call(
        _gauss_body,
        grid=grid,
        in_specs=[
            pl.BlockSpec((blk, DIM), lambda i: (i, 0)),
            pl.BlockSpec((DIM, L), lambda i: (0, 0)),
            pl.BlockSpec((DIM, L), lambda i: (0, 0)),
        ],
        out_specs=pl.BlockSpec((blk, L), lambda i: (i, 0)),
        out_shape=jax.ShapeDtypeStruct((N_EDGES, L), jnp.float32),
    )(edge_vectors, muT, sgT)


def _lane_bcast(v, lane):
    idx = jnp.full((L, 1), lane, dtype=jnp.int32)
    dn = lax.GatherDimensionNumbers(offset_dims=(),
                                    collapsed_slice_dims=(0,),
                                    start_index_map=(0,))
    return lax.gather(v, idx, dn, (1,),
                      mode=lax.GatherScatterMode.PROMISE_IN_BOUNDS)


def _sc_edge_body_v2(xg_hbm, g_hbm, src_hbm, dst_hbm, zero_hbm, out_hbm,
                     cnt_hbm, acc, src_c, dst_s, dst_c, g_c, rows_a, rows_b,
                     msg_c, cnt_v, sem):
    c = lax.axis_index("c")
    s = lax.axis_index("s")
    wid = s * NC + c
    rps = 624
    tail = N_NODES - NS * rps
    pltpu.sync_copy(zero_hbm.at[pl.ds(s * rps, rps)],
                    acc.at[pl.ds(s * rps, rps)])

    @pl.when(s == 0)
    def _zero_tail():
        pltpu.sync_copy(zero_hbm.at[pl.ds(NS * rps, tail)],
                        acc.at[pl.ds(NS * rps, tail)])

    zeros16 = jnp.zeros((L,), jnp.float32)
    ones16 = jnp.ones((L,), jnp.float32)
    lane0 = lax.iota(jnp.int32, L) == 0

    def zero_cnt(i, carry):
        cnt_v[pl.ds(i * L, L)] = zeros16
        return carry

    lax.fori_loop(0, N_NODES // L, zero_cnt, 0)
    plsc.subcore_barrier()

    rows = (rows_a, rows_b)
    NB = len(rows)
    ebase = wid * EPW

    def chunk_body(ic, carry):
        cb = ebase + ic * CE
        pltpu.sync_copy(src_hbm.at[pl.ds(cb, CE)], src_c)
        pltpu.sync_copy(dst_hbm.at[pl.ds(cb, CE)], dst_s)
        pltpu.sync_copy(dst_hbm.at[pl.ds(cb, CE)], dst_c.at[pl.ds(0, CE)])
        pltpu.sync_copy(g_hbm.at[pl.ds(cb, CE)], g_c)

        copies = [None] * NB
        for p in range(NB - 1):
            copies[p] = pltpu.async_copy(
                xg_hbm.at[src_c.at[pl.ds(p * BE, BE)]], rows[p], sem)
        for b in range(CH):
            buf = b % NB
            copies[buf].wait()
            nb = b + NB - 1
            if nb < CH:
                copies[nb % NB] = pltpu.async_copy(
                    xg_hbm.at[src_c.at[pl.ds(nb * BE, BE)]],
                    rows[nb % NB], sem)
            rows_v = rows[buf]
            dvec = dst_c[pl.ds((b // 2) * L, L)]

            def edge(e, carry3, _b=b, _rows=rows_v, _dvec=dvec):
                gv = g_c[_b * BE + e, :]
                accs = [jnp.zeros((L,), jnp.float32) for _ in range(8)]
                for k in range(K):
                    gk = _lane_bcast(gv, k)
                    for j in range(8):
                        accs[j] = accs[j] + gk * _rows[e, pl.ds(k * 128 + j * L, L)]
                for j in range(8):
                    msg_c[_b * BE + e, pl.ds(j * L, L)] = accs[j]
                dlane = _lane_bcast(_dvec, (_b % 2) * BE + e)
                plsc.addupdate_scatter(cnt_v, [dlane], ones16, mask=lane0)
                return carry3

            lax.fori_loop(0, BE, edge, 0)
        pltpu.sync_copy(msg_c, acc.at[dst_s], add=True)
        return carry

    lax.fori_loop(0, NCH, chunk_body, 0)
    plsc.subcore_barrier()
    pltpu.sync_copy(acc.at[pl.ds(s * rps, rps)],
                    out_hbm.at[c, pl.ds(s * rps, rps)])

    @pl.when(s == 0)
    def _out_tail():
        pltpu.sync_copy(acc.at[pl.ds(NS * rps, tail)],
                        out_hbm.at[c, pl.ds(NS * rps, tail)])

    pltpu.sync_copy(cnt_v, cnt_hbm.at[wid])


def _sc_edge(xg, gauss, src, dst, zero):
    mesh = plsc.VectorSubcoreMesh(core_axis_name="c", subcore_axis_name="s")
    fn = pl.kernel(
        _sc_edge_body_v2,
        out_type=(
            jax.ShapeDtypeStruct((NC, N_NODES, OUT_CH), jnp.float32),
            jax.ShapeDtypeStruct((NW, N_NODES), jnp.float32),
        ),
        mesh=mesh,
        compiler_params=pltpu.CompilerParams(needs_layout_passes=False),
        scratch_types=[
            pltpu.VMEM_SHARED((N_NODES, OUT_CH), jnp.float32),
            pltpu.VMEM((CE,), jnp.int32),
            pltpu.VMEM((CE,), jnp.int32),
            pltpu.VMEM((CE,), jnp.int32),
            pltpu.VMEM((CE, L), jnp.float32),
            pltpu.VMEM((BE, KW), jnp.float32),
            pltpu.VMEM((BE, KW), jnp.float32),
            pltpu.VMEM((CE, OUT_CH), jnp.float32),
            pltpu.VMEM((N_NODES,), jnp.float32),
            pltpu.SemaphoreType.DMA,
        ],
    )
    return fn(xg, gauss, src, dst, zero)


def _combine_body(acc_ref, cnt_ref, x_ref, rw_ref, b_ref, o_ref):
    sacc = acc_ref[0] + acc_ref[1]
    csum = jnp.sum(cnt_ref[...], axis=1, keepdims=True)
    mean = sacc / jnp.maximum(csum, 1.0)
    root = jnp.dot(x_ref[...], rw_ref[...], preferred_element_type=jnp.float32)
    o_ref[...] = mean + root + b_ref[...]


def _combine(acc, cnt, x, root_weight, bias2d):
    blk = 1000
    grid = (N_NODES // blk,)
    return pl.pallas_call(
        _combine_body,
        grid=grid,
        in_specs=[
            pl.BlockSpec((NC, blk, OUT_CH), lambda i: (0, i, 0)),
            pl.BlockSpec((blk, NW), lambda i: (i, 0)),
            pl.BlockSpec((blk, IN_CH), lambda i: (i, 0)),
            pl.BlockSpec((IN_CH, OUT_CH), lambda i: (0, 0)),
            pl.BlockSpec((1, OUT_CH), lambda i: (0, 0)),
        ],
        out_specs=pl.BlockSpec((blk, OUT_CH), lambda i: (i, 0)),
        out_shape=jax.ShapeDtypeStruct((N_NODES, OUT_CH), jnp.float32),
    )(acc, cnt, x, root_weight, bias2d)


def kernel(x, edge_index, edge_vectors, g_weight, mu, sigma, root_weight, bias):
    src = edge_index[0].astype(jnp.int32)
    dst = edge_index[1].astype(jnp.int32)
    mu16 = jnp.concatenate([mu, jnp.full((L - K, DIM), 1e16, mu.dtype)], axis=0)
    sg16 = jnp.concatenate([sigma, jnp.ones((L - K, DIM), sigma.dtype)], axis=0)
    muT = mu16.T
    sgT = sg16.T
    zero = jnp.zeros((N_NODES, OUT_CH), jnp.float32)
    bias2d = bias.reshape(1, OUT_CH)

    xg = _xg_matmul(x, g_weight)
    gauss = _gauss_weights(edge_vectors, muT, sgT)
    acc, cnt = _sc_edge(xg, gauss, src, dst, zero)
    return _combine(acc, cnt.T, x, root_weight, bias2d)
